# baseline (speedup 1.0000x reference)
import functools

import jax
import jax.numpy as jnp
from jax import lax
from jax.experimental import pallas as pl
from jax.experimental.pallas import tpu as pltpu
from jax.experimental.pallas import tpu_sc as plsc

N = 10000
E = 320000
D = 128
NH = 4
HD = D // NH

NSUB = 16
EPW = E // NSUB
ROWS = 80
NPAD = ROWS * 128
RPW = ROWS // NSUB
VPW = RPW * 128
NT = E // 128
TPW = NT // NSUB + 1


def _rsqrt_newton(v):
    xi = lax.bitcast_convert_type(v, jnp.int32)
    yi = jnp.int32(0x5F3759DF) - lax.shift_right_logical(xi, 1)
    y = lax.bitcast_convert_type(yi, jnp.float32)
    for _ in range(3):
        y = y * (1.5 - 0.5 * v * y * y)
    return y


def _slot_reduce(sh_all, my0, stage, stg2, sem_r0, sem_r1, acc):
    bufs = (stage, stg2)
    sems = (sem_r0, sem_r1)
    pltpu.async_copy(sh_all.at[pl.ds(my0, VPW)], stage, sem_r0)
    for w in range(NSUB):
        cur, nxt = bufs[w % 2], bufs[(w + 1) % 2]
        pltpu.make_async_copy(sh_all.at[pl.ds(w * NPAD + my0, VPW)], cur,
                              sems[w % 2]).wait()
        if w + 1 < NSUB:
            pltpu.async_copy(sh_all.at[pl.ds((w + 1) * NPAD + my0, VPW)],
                             nxt, sems[(w + 1) % 2])
        if w == 0:
            @plsc.parallel_loop(0, VPW, 16, unroll=8)
            def _(j):
                acc[pl.ds(j, 16)] = cur[pl.ds(j, 16)]
        else:
            @plsc.parallel_loop(0, VPW, 16, unroll=8)
            def _(j):
                acc[pl.ds(j, 16)] = acc[pl.ds(j, 16)] + cur[pl.ds(j, 16)]


def _sc_body(es_hbm, ed_hbm, cs_hbm, cd_hbm, out_e_hbm, out_c_hbm, cnt, tbuf, ebuf_s,
             ebuf_d, stage, stg2, dbuf, tacc, sh_all, sh_dinv, sem_s, sem_d,
             sem_r0, sem_r1):
    c = lax.axis_index("c")
    s = lax.axis_index("s")
    my0 = s * VPW

    e0 = s * EPW

    @pl.when(c == 0)
    def _():
        pltpu.async_copy(ed_hbm.at[pl.ds(e0, EPW)], ebuf_d, sem_d)
        pltpu.async_copy(es_hbm.at[pl.ds(e0, EPW)], ebuf_s, sem_s)

    @pl.when(c == 1)
    def _():
        pltpu.async_copy(cd_hbm.at[pl.ds(e0, EPW)], ebuf_d, sem_d)
        pltpu.async_copy(cs_hbm.at[pl.ds(e0, EPW)], ebuf_s, sem_s)

    h_d = pltpu.make_async_copy(ed_hbm.at[pl.ds(e0, EPW)], ebuf_d,
                                sem_d)
    h_s = pltpu.make_async_copy(es_hbm.at[pl.ds(e0, EPW)], ebuf_s,
                                sem_s)

    zeros16 = jnp.zeros((16,), jnp.float32)
    ones16 = jnp.ones((16,), jnp.float32)

    @plsc.parallel_loop(0, NPAD, 16, unroll=8)
    def _(i):
        cnt[pl.ds(i, 16)] = zeros16
        tbuf[pl.ds(i, 16)] = zeros16

    lane = lax.iota(jnp.int32, 16)
    h_d.wait()

    @plsc.parallel_loop(0, EPW, 16, unroll=8)
    def _(i):
        d16 = ebuf_d[pl.ds(i, 16)]
        plsc.addupdate_scatter(cnt, (d16,), ones16)

    pltpu.sync_copy(cnt, sh_all.at[pl.ds(s * NPAD, NPAD)])
    plsc.subcore_barrier()
    _slot_reduce(sh_all, my0, stage, stg2, sem_r0, sem_r1, dbuf)

    @plsc.parallel_loop(0, VPW, 16, unroll=4)
    def _(j):
        v = dbuf[pl.ds(j, 16)] + 1.0
        dbuf[pl.ds(j, 16)] = _rsqrt_newton(v)

    pltpu.sync_copy(dbuf, sh_dinv.at[pl.ds(my0, VPW)])
    plsc.subcore_barrier()
    pltpu.sync_copy(sh_dinv, cnt)

    h_s.wait()

    @plsc.parallel_loop(0, EPW, 16, unroll=8)
    def _(i):
        d16 = ebuf_d[pl.ds(i, 16)]
        s16 = ebuf_s[pl.ds(i, 16)]
        g = plsc.load_gather(cnt, (d16,))
        plsc.addupdate_scatter(tbuf, (s16,), g)

    pltpu.sync_copy(tbuf, sh_all.at[pl.ds(s * NPAD, NPAD)])
    plsc.subcore_barrier()
    _slot_reduce(sh_all, my0, stage, stg2, sem_r0, sem_r1, tacc)

    for j in range(VPW // 16):
        dv = dbuf[pl.ds(j * 16, 16)]
        tv = tacc[pl.ds(j * 16, 16)]
        wv = dv * (tv + dv)
        gi = my0 + j * 16 + lane
        wv = jnp.where(gi < N, wv, 0.0)
        tacc[pl.ds(j * 16, 16)] = wv

    @pl.when(c == 0)
    def _():
        pltpu.sync_copy(tacc, out_e_hbm.at[0, pl.ds(my0, VPW)])

    @pl.when(c == 1)
    def _():
        pltpu.sync_copy(tacc, out_c_hbm.at[0, pl.ds(my0, VPW)])


@functools.cache
def _sc_wvec():
    return functools.partial(
        pl.kernel,
        out_type=(jax.ShapeDtypeStruct((1, NPAD), jnp.float32),
                  jax.ShapeDtypeStruct((1, NPAD), jnp.float32)),
        mesh=plsc.VectorSubcoreMesh(core_axis_name="c", subcore_axis_name="s"),
        compiler_params=pltpu.CompilerParams(needs_layout_passes=False),
        scratch_types=[
            pltpu.VMEM((NPAD,), jnp.float32),
            pltpu.VMEM((NPAD,), jnp.float32),
            pltpu.VMEM((EPW,), jnp.int32),
            pltpu.VMEM((EPW,), jnp.int32),
            pltpu.VMEM((VPW,), jnp.float32),
            pltpu.VMEM((VPW,), jnp.float32),
            pltpu.VMEM((VPW,), jnp.float32),
            pltpu.VMEM((VPW,), jnp.float32),
            pltpu.VMEM_SHARED((NSUB * NPAD,), jnp.float32),
            pltpu.VMEM_SHARED((NPAD,), jnp.float32),
            pltpu.SemaphoreType.DMA,
            pltpu.SemaphoreType.DMA,
            pltpu.SemaphoreType.DMA,
            pltpu.SemaphoreType.DMA,
        ],
    )(_sc_body)


def _dgt(a, w):
    return lax.dot_general(a, w, (((1,), (1,)), ((), ())),
                           preferred_element_type=jnp.float32)


def _dg(a, b):
    return lax.dot_general(a, b, (((1,), (0,)), ((), ())),
                           preferred_element_type=jnp.float32)


def _tc_body(wv_e, wv_c, xe, xc, W1, b1, W2, b2, Wq, bq, Wk, bk, Wv_, bv,
             Wo, bo, g, b2_, Wout, bout, o_ref):
    inv_n = jnp.float32(1.0 / N)
    se = _dg(wv_e[...][:, :N], xe[...]) * inv_n
    sc = _dg(wv_c[...][:, :N], xc[...]) * inv_n
    pe = _dgt(se, W1[...]) + b1[...]
    pc = _dgt(sc, W2[...]) + b2[...]
    feats = jnp.concatenate([pe, pc], axis=0)

    q = _dgt(feats, Wq[...]) + bq[...]
    k = _dgt(feats, Wk[...]) + bk[...]
    v = _dgt(feats, Wv_[...]) + bv[...]
    outs = []
    scale = jnp.float32(1.0 / (HD ** 0.5))
    for h in range(NH):
        qh = q[:, h * HD:(h + 1) * HD]
        kh = k[:, h * HD:(h + 1) * HD]
        vh = v[:, h * HD:(h + 1) * HD]
        att = _dgt(qh, kh) * scale
        att = att - jnp.max(att, axis=-1, keepdims=True)
        ea = jnp.exp(att)
        att = ea / jnp.sum(ea, axis=-1, keepdims=True)
        outs.append(_dg(att, vh))
    o = jnp.concatenate(outs, axis=1)
    attended = _dgt(o, Wo[...]) + bo[...]

    xres = attended + feats
    m = jnp.mean(xres, axis=-1, keepdims=True)
    var = jnp.mean((xres - m) ** 2, axis=-1, keepdims=True)
    ln = (xres - m) / jnp.sqrt(var + 1e-5) * g[...] + b2_[...]
    fused = jnp.mean(ln, axis=0, keepdims=True)
    o_ref[...] = _dgt(fused, Wout[...]) + bout[...]


_tc_tail = pl.pallas_call(
    _tc_body,
    out_shape=jax.ShapeDtypeStruct((1, D), jnp.float32),
)


def kernel(x_ego, ego_edge_index, x_cut, cut_edge_index, W1, b1, W2, b2,
           Wq, bq, Wk, bk, Wv, bv, Wo, bo, ln_g, ln_b, Wout, bout):
    ee = ego_edge_index.astype(jnp.int32)
    ce = cut_edge_index.astype(jnp.int32)
    wv_e, wv_c = _sc_wvec()(ee[0], ee[1], ce[0], ce[1])
    r = lambda a: a.reshape(1, D)
    return _tc_tail(wv_e, wv_c, x_ego, x_cut, W1, r(b1), W2, r(b2), Wq, r(bq),
                    Wk, r(bk), Wv, r(bv), Wo, r(bo), r(ln_g), r(ln_b),
                    Wout, r(bout))


# speedup vs baseline: 1.2035x; 1.2035x over previous
import functools

import jax
import jax.numpy as jnp
from jax import lax
from jax.experimental import pallas as pl
from jax.experimental.pallas import tpu as pltpu
from jax.experimental.pallas import tpu_sc as plsc

N = 10000
E = 320000
D = 128
NH = 4
HD = D // NH

NSUB = 16
EPW = E // NSUB
ROWS = 80
NPAD = ROWS * 128
RPW = ROWS // NSUB
VPW = RPW * 128
NT = E // 128
TPW = NT // NSUB + 1


def _rsqrt_newton(v):
    xi = lax.bitcast_convert_type(v, jnp.int32)
    yi = jnp.int32(0x5F3759DF) - lax.shift_right_logical(xi, 1)
    y = lax.bitcast_convert_type(yi, jnp.float32)
    for _ in range(3):
        y = y * (1.5 - 0.5 * v * y * y)
    return y


def _slot_reduce(sh_all, my0, stage, stg2, sem_r0, sem_r1, acc):
    bufs = (stage, stg2)
    sems = (sem_r0, sem_r1)
    pltpu.async_copy(sh_all.at[pl.ds(my0, VPW)], stage, sem_r0)
    for w in range(NSUB):
        cur, nxt = bufs[w % 2], bufs[(w + 1) % 2]
        pltpu.make_async_copy(sh_all.at[pl.ds(w * NPAD + my0, VPW)], cur,
                              sems[w % 2]).wait()
        if w + 1 < NSUB:
            pltpu.async_copy(sh_all.at[pl.ds((w + 1) * NPAD + my0, VPW)],
                             nxt, sems[(w + 1) % 2])
        if w == 0:
            @plsc.parallel_loop(0, VPW, 16, unroll=8)
            def _(j):
                acc[pl.ds(j, 16)] = cur[pl.ds(j, 16)]
        else:
            @plsc.parallel_loop(0, VPW, 16, unroll=8)
            def _(j):
                acc[pl.ds(j, 16)] = acc[pl.ds(j, 16)] + cur[pl.ds(j, 16)]


def _sc_body(ego_hbm, cut_hbm, out_e_hbm, out_c_hbm, cnt, tbuf, ebuf_s,
             ebuf_d, stage, stg2, dbuf, tacc, sh_all, sh_dinv,
             sem_s, sem_d, sem_r0, sem_r1):
    c = lax.axis_index("c")
    s = lax.axis_index("s")
    my0 = s * VPW

    e0 = s * EPW

    @pl.when(c == 0)
    def _():
        pltpu.async_copy(ego_hbm.at[1, pl.ds(e0, EPW)], ebuf_d, sem_d)
        pltpu.async_copy(ego_hbm.at[0, pl.ds(e0, EPW)], ebuf_s, sem_s)

    @pl.when(c == 1)
    def _():
        pltpu.async_copy(cut_hbm.at[1, pl.ds(e0, EPW)], ebuf_d, sem_d)
        pltpu.async_copy(cut_hbm.at[0, pl.ds(e0, EPW)], ebuf_s, sem_s)

    h_d = pltpu.make_async_copy(ego_hbm.at[1, pl.ds(e0, EPW)], ebuf_d, sem_d)
    h_s = pltpu.make_async_copy(ego_hbm.at[0, pl.ds(e0, EPW)], ebuf_s, sem_s)

    zeros16 = jnp.zeros((16,), jnp.float32)
    ones16 = jnp.ones((16,), jnp.float32)

    @plsc.parallel_loop(0, NPAD, 16, unroll=8)
    def _(i):
        cnt[pl.ds(i, 16)] = zeros16
        tbuf[pl.ds(i, 16)] = zeros16

    lane = lax.iota(jnp.int32, 16)
    h_d.wait()

    @plsc.parallel_loop(0, EPW, 16, unroll=8)
    def _(i):
        d16 = ebuf_d[pl.ds(i, 16)]
        plsc.addupdate_scatter(cnt, (d16,), ones16)

    pltpu.sync_copy(cnt, sh_all.at[pl.ds(s * NPAD, NPAD)])
    plsc.subcore_barrier()
    _slot_reduce(sh_all, my0, stage, stg2, sem_r0, sem_r1, dbuf)

    @plsc.parallel_loop(0, VPW, 16, unroll=4)
    def _(j):
        v = dbuf[pl.ds(j, 16)] + 1.0
        dbuf[pl.ds(j, 16)] = _rsqrt_newton(v)

    pltpu.sync_copy(dbuf, sh_dinv.at[pl.ds(my0, VPW)])
    plsc.subcore_barrier()
    pltpu.sync_copy(sh_dinv, cnt)

    h_s.wait()

    @plsc.parallel_loop(0, EPW, 16, unroll=8)
    def _(i):
        d16 = ebuf_d[pl.ds(i, 16)]
        s16 = ebuf_s[pl.ds(i, 16)]
        g = plsc.load_gather(cnt, (d16,))
        plsc.addupdate_scatter(tbuf, (s16,), g)

    pltpu.sync_copy(tbuf, sh_all.at[pl.ds(s * NPAD, NPAD)])
    plsc.subcore_barrier()
    _slot_reduce(sh_all, my0, stage, stg2, sem_r0, sem_r1, tacc)

    for j in range(VPW // 16):
        dv = dbuf[pl.ds(j * 16, 16)]
        tv = tacc[pl.ds(j * 16, 16)]
        wv = dv * (tv + dv)
        gi = my0 + j * 16 + lane
        wv = jnp.where(gi < N, wv, 0.0)
        tacc[pl.ds(j * 16, 16)] = wv

    @pl.when(c == 0)
    def _():
        pltpu.sync_copy(tacc, out_e_hbm.at[0, pl.ds(my0, VPW)])

    @pl.when(c == 1)
    def _():
        pltpu.sync_copy(tacc, out_c_hbm.at[0, pl.ds(my0, VPW)])


@functools.cache
def _sc_wvec():
    return functools.partial(
        pl.kernel,
        out_type=(jax.ShapeDtypeStruct((1, NPAD), jnp.float32),
                  jax.ShapeDtypeStruct((1, NPAD), jnp.float32)),
        mesh=plsc.VectorSubcoreMesh(core_axis_name="c", subcore_axis_name="s"),
        compiler_params=pltpu.CompilerParams(needs_layout_passes=False,
                                             use_tc_tiling_on_sc=False),
        scratch_types=[
            pltpu.VMEM((NPAD,), jnp.float32),
            pltpu.VMEM((NPAD,), jnp.float32),
            pltpu.VMEM((EPW,), jnp.int32),
            pltpu.VMEM((EPW,), jnp.int32),
            pltpu.VMEM((VPW,), jnp.float32),
            pltpu.VMEM((VPW,), jnp.float32),
            pltpu.VMEM((VPW,), jnp.float32),
            pltpu.VMEM((VPW,), jnp.float32),
            pltpu.VMEM_SHARED((NSUB * NPAD,), jnp.float32),
            pltpu.VMEM_SHARED((NPAD,), jnp.float32),
            pltpu.SemaphoreType.DMA,
            pltpu.SemaphoreType.DMA,
            pltpu.SemaphoreType.DMA,
            pltpu.SemaphoreType.DMA,
        ],
    )(_sc_body)


def _dgt(a, w):
    return lax.dot_general(a, w, (((1,), (1,)), ((), ())),
                           preferred_element_type=jnp.float32)


def _dg(a, b):
    return lax.dot_general(a, b, (((1,), (0,)), ((), ())),
                           preferred_element_type=jnp.float32)


def _tc_body(wv_e, wv_c, xe, xc, W1, b1, W2, b2, Wq, bq, Wk, bk, Wv_, bv,
             Wo, bo, g, b2_, Wout, bout, o_ref):
    inv_n = jnp.float32(1.0 / N)
    se = _dg(wv_e[...][:, :N], xe[...]) * inv_n
    sc = _dg(wv_c[...][:, :N], xc[...]) * inv_n
    pe = _dgt(se, W1[...]) + b1[...]
    pc = _dgt(sc, W2[...]) + b2[...]
    feats = jnp.concatenate([pe, pc], axis=0)

    q = _dgt(feats, Wq[...]) + bq[...]
    k = _dgt(feats, Wk[...]) + bk[...]
    v = _dgt(feats, Wv_[...]) + bv[...]
    outs = []
    scale = jnp.float32(1.0 / (HD ** 0.5))
    for h in range(NH):
        qh = q[:, h * HD:(h + 1) * HD]
        kh = k[:, h * HD:(h + 1) * HD]
        vh = v[:, h * HD:(h + 1) * HD]
        att = _dgt(qh, kh) * scale
        att = att - jnp.max(att, axis=-1, keepdims=True)
        ea = jnp.exp(att)
        att = ea / jnp.sum(ea, axis=-1, keepdims=True)
        outs.append(_dg(att, vh))
    o = jnp.concatenate(outs, axis=1)
    attended = _dgt(o, Wo[...]) + bo[...]

    xres = attended + feats
    m = jnp.mean(xres, axis=-1, keepdims=True)
    var = jnp.mean((xres - m) ** 2, axis=-1, keepdims=True)
    ln = (xres - m) / jnp.sqrt(var + 1e-5) * g[...] + b2_[...]
    fused = jnp.mean(ln, axis=0, keepdims=True)
    o_ref[...] = _dgt(fused, Wout[...]) + bout[...]


_tc_tail = pl.pallas_call(
    _tc_body,
    out_shape=jax.ShapeDtypeStruct((1, D), jnp.float32),
)


def kernel(x_ego, ego_edge_index, x_cut, cut_edge_index, W1, b1, W2, b2,
           Wq, bq, Wk, bk, Wv, bv, Wo, bo, ln_g, ln_b, Wout, bout):
    wv_e, wv_c = _sc_wvec()(ego_edge_index.astype(jnp.int32),
                            cut_edge_index.astype(jnp.int32))
    r = lambda a: a.reshape(1, D)
    return _tc_tail(wv_e, wv_c, x_ego, x_cut, W1, r(b1), W2, r(b2), Wq, r(bq),
                    Wk, r(bk), Wv, r(bv), Wo, r(bo), r(ln_g), r(ln_b),
                    Wout, r(bout))
